# baseline (speedup 1.0000x reference)
import jax
import jax.numpy as jnp
from jax.experimental import pallas as pl
from jax.experimental.pallas import tpu as pltpu

_CHUNK = 5000
_NBUF = 4


def _mm_bias_kernel(x_hbm, w_ref, b_ref, o_hbm, x_buf, o_buf, in_sems, out_sems):
    n = x_hbm.shape[0]
    nchunk = n // _CHUNK

    def in_copy(i, s):
        return pltpu.make_async_copy(
            x_hbm.at[pl.ds(i * _CHUNK, _CHUNK), :], x_buf.at[s], in_sems.at[s]
        )

    def out_copy(i, s):
        return pltpu.make_async_copy(
            o_buf.at[s], o_hbm.at[pl.ds(i * _CHUNK, _CHUNK), :], out_sems.at[s]
        )

    for s in range(_NBUF):
        in_copy(s, s).start()

    w = w_ref[...]
    b = b_ref[...]

    for i in range(nchunk):
        s = i % _NBUF
        in_copy(i, s).wait()
        o = jnp.dot(x_buf[s], w, preferred_element_type=jnp.float32) + b
        if i >= _NBUF:
            out_copy(i - _NBUF, s).wait()
        o_buf[s] = o
        out_copy(i, s).start()
        if i + _NBUF < nchunk:
            in_copy(i + _NBUF, s).start()

    for i in range(nchunk - _NBUF, nchunk):
        out_copy(i, i % _NBUF).wait()


def kernel(input, kernel, bias):
    n, cin = input.shape
    cout = kernel.shape[1]
    return pl.pallas_call(
        _mm_bias_kernel,
        in_specs=[
            pl.BlockSpec(memory_space=pltpu.MemorySpace.HBM),
            pl.BlockSpec((cin, cout), lambda: (0, 0)),
            pl.BlockSpec((1, cout), lambda: (0, 0)),
        ],
        out_specs=pl.BlockSpec(memory_space=pltpu.MemorySpace.HBM),
        out_shape=jax.ShapeDtypeStruct((n, cout), jnp.float32),
        scratch_shapes=[
            pltpu.VMEM((_NBUF, _CHUNK, cin), jnp.float32),
            pltpu.VMEM((_NBUF, _CHUNK, cout), jnp.float32),
            pltpu.SemaphoreType.DMA((_NBUF,)),
            pltpu.SemaphoreType.DMA((_NBUF,)),
        ],
    )(input, kernel, bias)


# speedup vs baseline: 1.1161x; 1.1161x over previous
import jax
import jax.numpy as jnp
from jax.experimental import pallas as pl
from jax.experimental.pallas import tpu as pltpu

_CHUNK = 10000
_NBUF = 3


def _mm_bias_kernel(x_hbm, w_ref, b_ref, o_hbm, x_buf, o_buf, in_sems, out_sems):
    n = x_hbm.shape[0]
    nchunk = n // _CHUNK

    def in_copy(i, s):
        return pltpu.make_async_copy(
            x_hbm.at[pl.ds(i * _CHUNK, _CHUNK), :], x_buf.at[s], in_sems.at[s]
        )

    def out_copy(i, s):
        return pltpu.make_async_copy(
            o_buf.at[s], o_hbm.at[pl.ds(i * _CHUNK, _CHUNK), :], out_sems.at[s]
        )

    for s in range(_NBUF):
        in_copy(s, s).start()

    w = w_ref[...]
    b = b_ref[...]

    for i in range(nchunk):
        s = i % _NBUF
        in_copy(i, s).wait()
        o = jnp.dot(x_buf[s], w, preferred_element_type=jnp.float32) + b
        if i >= _NBUF:
            out_copy(i - _NBUF, s).wait()
        o_buf[s] = o
        out_copy(i, s).start()
        if i + _NBUF < nchunk:
            in_copy(i + _NBUF, s).start()

    for i in range(nchunk - _NBUF, nchunk):
        out_copy(i, i % _NBUF).wait()


def kernel(input, kernel, bias):
    n, cin = input.shape
    cout = kernel.shape[1]
    return pl.pallas_call(
        _mm_bias_kernel,
        in_specs=[
            pl.BlockSpec(memory_space=pltpu.MemorySpace.HBM),
            pl.BlockSpec((cin, cout), lambda: (0, 0)),
            pl.BlockSpec((1, cout), lambda: (0, 0)),
        ],
        out_specs=pl.BlockSpec(memory_space=pltpu.MemorySpace.HBM),
        out_shape=jax.ShapeDtypeStruct((n, cout), jnp.float32),
        scratch_shapes=[
            pltpu.VMEM((_NBUF, _CHUNK, cin), jnp.float32),
            pltpu.VMEM((_NBUF, _CHUNK, cout), jnp.float32),
            pltpu.SemaphoreType.DMA((_NBUF,)),
            pltpu.SemaphoreType.DMA((_NBUF,)),
        ],
    )(input, kernel, bias)
